# Initial kernel scaffold; baseline (speedup 1.0000x reference)
import jax
import jax.numpy as jnp
from jax.experimental import pallas as pl


def kernel(x, knn, W, bn_weight, bn_bias):
    raise NotImplementedError("write your pallas kernel here")


# speedup vs baseline: 10.2144x; 10.2144x over previous
import functools

import jax
import jax.numpy as jnp
from jax import lax
from jax.experimental import pallas as pl
from jax.experimental.pallas import tpu as pltpu
from jax.experimental.pallas import tpu_sc as plsc

Bn, N, K = 2, 10000, 16
D = 128
BN_EPS = 1e-5
NP = Bn * N
NC, NS, L = 2, 16, 16
NW = NC * NS
PW = 640
NPAD = NW * PW
C = 8
IDXC = C * K
NCH = PW // C
NJ = D // L


def _mm_body(x_ref, w_ref, o_ref):
    o_ref[...] = lax.dot_general(
        x_ref[...], w_ref[...], (((1,), (1,)), ((), ())),
        preferred_element_type=jnp.float32)


def _bn_body(y_ref, p_ref, w_ref, b_ref, o_ref):
    ps = jnp.sum(p_ref[...], axis=0).reshape(2, D)
    mean = ps[0:1, :] * (1.0 / NP)
    msq = ps[1:2, :] * (1.0 / NP)
    var = msq - mean * mean
    scale = w_ref[...] * lax.rsqrt(var + BN_EPS)
    shift = b_ref[...] - mean * scale
    o_ref[...] = y_ref[...] * scale + shift


def _sc_body(h_hbm, idx_hbm, y_hbm, part_hbm,
             idx_v, rows_a, rows_b, cent_a, cent_b, out_a, out_b, stats_v,
             sg_a, sg_b, sc_a, sc_b, so_a, so_b):
    c = lax.axis_index("c")
    s = lax.axis_index("s")
    wid = s * NC + c
    base = pl.multiple_of(wid * PW, PW)

    pltpu.sync_copy(idx_hbm.at[pl.ds(pl.multiple_of(base * K, 8), PW * K)],
                    idx_v)
    for i in range(2 * D // L):
        stats_v[pl.ds(i * L, L)] = jnp.zeros((L,), jnp.float32)

    def g_start(g, rows, sem):
        iof = pl.multiple_of(g * IDXC, 8)
        pltpu.async_copy(h_hbm.at[idx_v.at[pl.ds(iof, IDXC)]], rows, sem)

    def c_start(g, cent, sem):
        prow = jnp.minimum(base + g * C, NP - C)
        pltpu.async_copy(h_hbm.at[pl.ds(prow, C)], cent, sem)

    def s_start(g, outb, sem):
        pltpu.async_copy(outb, y_hbm.at[pl.ds(base + g * C, C)], sem)

    def in_wait(dst, sem, nrows):
        pltpu.make_async_copy(h_hbm.at[pl.ds(0, nrows)], dst, sem).wait()

    def out_wait(outb, sem):
        pltpu.make_async_copy(outb, y_hbm.at[pl.ds(0, C)], sem).wait()

    def compute(g, rows, cent, outb):
        prow = base + g * C
        mvec = jnp.broadcast_to(
            jnp.where(prow < NP, 1.0, 0.0).astype(jnp.float32), (L,))
        zeros8 = tuple(jnp.zeros((L,), jnp.float32) for _ in range(NJ))

        @pl.loop(0, C, init_carry=(zeros8, zeros8))
        def point_loop(p, carry):
            sums, sqs = carry
            r0 = p * K
            accs = [rows[r0, pl.ds(j * L, L)] for j in range(NJ)]
            for k in range(1, K):
                for j in range(NJ):
                    accs[j] = jnp.maximum(accs[j], rows[r0 + k, pl.ds(j * L, L)])
            new_sums, new_sqs = [], []
            for j in range(NJ):
                yv = accs[j] - cent[p, pl.ds(j * L, L)]
                outb[p, pl.ds(j * L, L)] = yv
                new_sums.append(sums[j] + yv)
                new_sqs.append(sqs[j] + yv * yv)
            return tuple(new_sums), tuple(new_sqs)

        sums, sqs = point_loop
        for j in range(NJ):
            stats_v[pl.ds(j * L, L)] = stats_v[pl.ds(j * L, L)] + sums[j] * mvec
            stats_v[pl.ds(D + j * L, L)] = (stats_v[pl.ds(D + j * L, L)]
                                            + sqs[j] * mvec)

    def do_chunk(g, rows, cent, outb, sg, sc_, so, first, last):
        in_wait(rows, sg, IDXC)
        in_wait(cent, sc_, C)
        if not first:
            out_wait(outb, so)
        compute(g, rows, cent, outb)
        s_start(g, outb, so)
        if not last:
            g_start(g + 2, rows, sg)
            c_start(g + 2, cent, sc_)

    buf_a = (rows_a, cent_a, out_a, sg_a, sc_a, so_a)
    buf_b = (rows_b, cent_b, out_b, sg_b, sc_b, so_b)

    g_start(0, rows_a, sg_a)
    c_start(0, cent_a, sc_a)
    g_start(1, rows_b, sg_b)
    c_start(1, cent_b, sc_b)
    do_chunk(0, *buf_a, first=True, last=False)
    do_chunk(1, *buf_b, first=True, last=False)

    @pl.loop(2, NCH - 2, step=2)
    def chunk_loop(g):
        do_chunk(g, *buf_a, first=False, last=False)
        do_chunk(g + 1, *buf_b, first=False, last=False)

    do_chunk(NCH - 2, *buf_a, first=False, last=True)
    do_chunk(NCH - 1, *buf_b, first=False, last=True)
    out_wait(out_a, so_a)
    out_wait(out_b, so_b)
    pltpu.sync_copy(stats_v, part_hbm.at[wid])


@functools.partial(
    pl.kernel,
    out_type=(jax.ShapeDtypeStruct((NPAD, D), jnp.float32),
              jax.ShapeDtypeStruct((NW, 2 * D), jnp.float32)),
    mesh=plsc.VectorSubcoreMesh(core_axis_name="c", subcore_axis_name="s",
                                num_cores=NC, num_subcores=NS),
    scratch_types=[
        pltpu.VMEM((PW * K,), jnp.int32),
        pltpu.VMEM((IDXC, D), jnp.float32),
        pltpu.VMEM((IDXC, D), jnp.float32),
        pltpu.VMEM((C, D), jnp.float32),
        pltpu.VMEM((C, D), jnp.float32),
        pltpu.VMEM((C, D), jnp.float32),
        pltpu.VMEM((C, D), jnp.float32),
        pltpu.VMEM((2 * D,), jnp.float32),
        pltpu.SemaphoreType.DMA,
        pltpu.SemaphoreType.DMA,
        pltpu.SemaphoreType.DMA,
        pltpu.SemaphoreType.DMA,
        pltpu.SemaphoreType.DMA,
        pltpu.SemaphoreType.DMA,
    ],
)
def _sc_gather_max(h_hbm, idx_hbm, y_hbm, part_hbm, *scratch):
    _sc_body(h_hbm, idx_hbm, y_hbm, part_hbm, *scratch)


def kernel(x, knn, W, bn_weight, bn_bias):
    x2 = x.reshape(NP, D)
    h = pl.pallas_call(
        _mm_body,
        grid=(10,),
        in_specs=[pl.BlockSpec((NP // 10, D), lambda i: (i, 0)),
                  pl.BlockSpec((D, D), lambda i: (0, 0))],
        out_specs=pl.BlockSpec((NP // 10, D), lambda i: (i, 0)),
        out_shape=jax.ShapeDtypeStruct((NP, D), jnp.float32),
    )(x2, W)

    offs = (jnp.arange(Bn, dtype=jnp.int32) * N).reshape(Bn, 1, 1)
    fidx = (knn.astype(jnp.int32) + offs).reshape(NP * K)
    fidx = jnp.concatenate(
        [fidx, jnp.zeros(((NPAD - NP) * K,), jnp.int32)])

    y_pad, partials = _sc_gather_max(h, fidx)

    out = pl.pallas_call(
        _bn_body,
        grid=(10,),
        in_specs=[pl.BlockSpec((NP // 10, D), lambda i: (i, 0)),
                  pl.BlockSpec((NW, 2 * D), lambda i: (0, 0)),
                  pl.BlockSpec((1, D), lambda i: (0, 0)),
                  pl.BlockSpec((1, D), lambda i: (0, 0))],
        out_specs=pl.BlockSpec((NP // 10, D), lambda i: (i, 0)),
        out_shape=jax.ShapeDtypeStruct((NP, D), jnp.float32),
    )(y_pad[:NP], partials, bn_weight.reshape(1, D), bn_bias.reshape(1, D))
    return out.reshape(Bn, N, D)
